# Initial kernel scaffold; baseline (speedup 1.0000x reference)
import jax
import jax.numpy as jnp
from jax.experimental import pallas as pl


def kernel(x, pos, edge_index, W_lin, W_src, W_dst, pos_w1, pos_b1, pos_w2, pos_b2, attn_w1, attn_b1, attn_w2, attn_b2, bn_gamma, bn_beta):
    raise NotImplementedError("write your pallas kernel here")


# speedup vs baseline: 12.8412x; 12.8412x over previous
import functools

import jax
import jax.numpy as jnp
from jax import lax
from jax.experimental import pallas as pl
from jax.experimental.pallas import tpu as pltpu
from jax.experimental.pallas import tpu_sc as plsc

N = 10000
D = 128
NB = 5
BLK = N // NB
NC = 2
NS = 16
CHUNK = 128
ACC_ROWS = 10240
GARBAGE = N

_HI = lax.Precision.HIGHEST


def _dotT(a, w):
    return lax.dot_general(a, w, (((1,), (1,)), ((), ())),
                           precision=_HI, preferred_element_type=jnp.float32)


def _k1_body(x_ref, p_ref, wlin_ref, wsrc_ref, pw1_ref, pw2_ref, pb1_ref,
             pb2_ref, aw1_ref, aw2_ref, H_ref, A_ref, B_ref, hmin_ref):
    x = x_ref[...]
    t = _dotT(p_ref[...], pw1_ref[...])
    q = _dotT(t, pw2_ref[...])
    u = _dotT(x, wsrc_ref[...]) + q
    H = _dotT(_dotT(u, aw1_ref[...]), aw2_ref[...])
    bp = _dotT(pb1_ref[...], pw2_ref[...]) + pb2_ref[...]
    H_ref[...] = H
    A_ref[...] = _dotT(x, wlin_ref[...]) - q
    B_ref[...] = q + bp
    hmin_ref[0] = jnp.min(H, axis=0, keepdims=True)


def _run_k1(x, pos_pad, W_lin, W_src, pw1_pad, pos_w2, pb1, pb2, attn_w1, attn_w2):
    full = lambda s: pl.BlockSpec(s, lambda i: (0, 0))
    row = pl.BlockSpec((BLK, D), lambda i: (i, 0))
    return pl.pallas_call(
        _k1_body,
        grid=(NB,),
        in_specs=[row, row, full((D, D)), full((D, D)), full((64, D)),
                  full((D, 64)), full((1, 64)), full((1, D)),
                  full((64, D)), full((D, 64))],
        out_specs=[row, row, row, pl.BlockSpec((1, 1, D), lambda i: (i, 0, 0))],
        out_shape=[jax.ShapeDtypeStruct((N, D), jnp.float32),
                   jax.ShapeDtypeStruct((N, D), jnp.float32),
                   jax.ShapeDtypeStruct((N, D), jnp.float32),
                   jax.ShapeDtypeStruct((NB, 1, D), jnp.float32)],
    )(x, pos_pad, W_lin, W_src, pw1_pad, pos_w2, pb1, pb2, attn_w1, attn_w2)


def _k2_body(hpart_ref, H_ref, A_ref, T_ref):
    hmin = jnp.min(hpart_ref[...], axis=0, keepdims=True)
    w = jnp.exp(hmin - H_ref[...])
    T_ref[0] = w
    T_ref[1] = w * A_ref[...]


def _run_k2(hpart, H, A):
    row = pl.BlockSpec((BLK, D), lambda i: (i, 0))
    return pl.pallas_call(
        _k2_body,
        grid=(NB,),
        in_specs=[pl.BlockSpec((NB, D), lambda i: (0, 0)), row, row],
        out_specs=pl.BlockSpec((2, BLK, D), lambda i: (0, i, 0)),
        out_shape=jax.ShapeDtypeStruct((2, N, D), jnp.float32),
    )(hpart, H, A)


CPG = 32


def _make_sc_kernel(n_groups):
    mesh = plsc.VectorSubcoreMesh(core_axis_name="c", subcore_axis_name="s",
                                  num_cores=NC, num_subcores=NS)
    rows_per_tile = ACC_ROWS // NS

    @functools.partial(
        pl.kernel,
        out_type=jax.ShapeDtypeStruct((NC, ACC_ROWS, D), jnp.float32),
        mesh=mesh,
        scratch_types=[
            pltpu.VMEM((CPG, CHUNK), jnp.int32),
            pltpu.VMEM((CPG, CHUNK), jnp.int32),
            pltpu.VMEM((CHUNK, D), jnp.float32),
            pltpu.MemorySpace.VMEM_SHARED((ACC_ROWS, D), jnp.float32),
            pltpu.SemaphoreType.DMA,
        ],
    )
    def sc_scatter(tall_hbm, srcc_hbm, dstr_hbm, acc_hbm,
                   idx_s, idx_d, buf, acc_sh, sem):
        c = lax.axis_index("c")
        s = lax.axis_index("s")

        def _z(i, carry):
            buf[i // 8, pl.ds((i % 8) * 16, 16)] = jnp.zeros((16,), jnp.float32)
            return carry
        lax.fori_loop(0, CHUNK * 8, _z, 0)
        base = s * rows_per_tile
        for k in range(rows_per_tile // CHUNK):
            pltpu.sync_copy(buf, acc_sh.at[pl.ds(base + k * CHUNK, CHUNK)])
        plsc.subcore_barrier()

        def _group(g, carry):
            pltpu.sync_copy(srcc_hbm.at[c, s, g], idx_s)
            pltpu.sync_copy(dstr_hbm.at[s, g], idx_d)

            def _edge_chunk(j, inner):
                pltpu.async_copy(tall_hbm.at[idx_s.at[j]], buf, sem).wait()
                pltpu.sync_copy(buf, acc_sh.at[idx_d.at[j]], add=True)
                return inner
            return lax.fori_loop(0, CPG, _edge_chunk, carry)
        lax.fori_loop(0, n_groups, _group, 0)
        plsc.subcore_barrier()

        pltpu.sync_copy(acc_sh.at[pl.ds(base, rows_per_tile)],
                        acc_hbm.at[c, pl.ds(base, rows_per_tile)])

    return sc_scatter


def _k3_body(a0_ref, a1_ref, t0_ref, t1_ref, b_ref, o_ref, ps_ref, pq_ref):
    denom = a0_ref[...] + t0_ref[...]
    numer = a1_ref[...] + t1_ref[...] + b_ref[...] * denom
    o = numer / (denom + 1e-16)
    o = jnp.where(o > 0, o, jnp.exp(o) - 1.0)
    o_ref[...] = o
    ps_ref[0] = jnp.sum(o, axis=0, keepdims=True)
    pq_ref[0] = jnp.sum(o * o, axis=0, keepdims=True)


def _run_k3(a0, a1, t0, t1, B):
    row = pl.BlockSpec((BLK, D), lambda i: (i, 0))
    return pl.pallas_call(
        _k3_body,
        grid=(NB,),
        in_specs=[row, row, row, row, row],
        out_specs=[row, pl.BlockSpec((1, 1, D), lambda i: (i, 0, 0)),
                   pl.BlockSpec((1, 1, D), lambda i: (i, 0, 0))],
        out_shape=[jax.ShapeDtypeStruct((N, D), jnp.float32),
                   jax.ShapeDtypeStruct((NB, 1, D), jnp.float32),
                   jax.ShapeDtypeStruct((NB, 1, D), jnp.float32)],
    )(a0, a1, t0, t1, B)


def _k4_body(o_ref, ps_ref, pq_ref, g_ref, bb_ref, out_ref):
    inv_n = 1.0 / N
    mean = jnp.sum(ps_ref[...], axis=0, keepdims=True) * inv_n
    var = jnp.sum(pq_ref[...], axis=0, keepdims=True) * inv_n - mean * mean
    scale = lax.rsqrt(var + 1e-5) * g_ref[...]
    out_ref[...] = (o_ref[...] - mean) * scale + bb_ref[...]


def _run_k4(o, ps, pq, gamma, beta):
    row = pl.BlockSpec((BLK, D), lambda i: (i, 0))
    part = pl.BlockSpec((NB, D), lambda i: (0, 0))
    vec = pl.BlockSpec((1, D), lambda i: (0, 0))
    return pl.pallas_call(
        _k4_body,
        grid=(NB,),
        in_specs=[row, part, part, vec, vec],
        out_specs=row,
        out_shape=jax.ShapeDtypeStruct((N, D), jnp.float32),
    )(o, ps, pq, gamma, beta)


def kernel(x, pos, edge_index, W_lin, W_src, W_dst,
           pos_w1, pos_b1, pos_w2, pos_b2,
           attn_w1, attn_b1, attn_w2, attn_b2,
           bn_gamma, bn_beta):
    pos_pad = jnp.pad(pos, ((0, 0), (0, D - pos.shape[1])))
    pw1_pad = jnp.pad(pos_w1, ((0, 0), (0, D - pos_w1.shape[1])))
    pb1 = pos_b1.reshape(1, -1)
    pb2 = pos_b2.reshape(1, -1)

    H, A, B, hpart = _run_k1(x, pos_pad, W_lin, W_src, pw1_pad, pos_w2,
                             pb1, pb2, attn_w1, attn_w2)
    T2 = _run_k2(hpart.reshape(NB, D), H, A)
    t_all = T2.reshape(2 * N, D)

    src, dst = edge_index[0], edge_index[1]
    e_in = src.shape[0]
    n_groups = -(-e_in // (NS * CPG * CHUNK))
    ept = n_groups * CPG * CHUNK
    pad = NS * ept - e_in
    dst_eff = jnp.where(src != dst, dst, GARBAGE)
    src_p = jnp.concatenate([src, jnp.zeros((pad,), jnp.int32)])
    dst_p = jnp.concatenate([dst_eff, jnp.full((pad,), GARBAGE, jnp.int32)])
    src_r = src_p.reshape(NS, n_groups, CPG, CHUNK)
    dst_r = dst_p.reshape(NS, n_groups, CPG, CHUNK)
    src_rc = jnp.stack([src_r, src_r + N])

    acc = _make_sc_kernel(n_groups)(t_all, src_rc, dst_r)

    o, ps, pq = _run_k3(acc[0, :N], acc[1, :N], T2[0], T2[1], B)
    return _run_k4(o, ps.reshape(NB, D), pq.reshape(NB, D),
                   bn_gamma.reshape(1, -1), bn_beta.reshape(1, -1))


# speedup vs baseline: 15.2886x; 1.1906x over previous
import functools

import jax
import jax.numpy as jnp
from jax import lax
from jax.experimental import pallas as pl
from jax.experimental.pallas import tpu as pltpu
from jax.experimental.pallas import tpu_sc as plsc

N = 10000
D = 128
NB = 5
BLK = N // NB
NC = 2
NS = 16
CHUNK = 128
ACC_ROWS = 10240
GARBAGE = N

_HI = lax.Precision.HIGHEST


def _dotT(a, w):
    return lax.dot_general(a, w, (((1,), (1,)), ((), ())),
                           precision=_HI, preferred_element_type=jnp.float32)


def _k1_body(x_ref, p_ref, wlin_ref, wsrc_ref, pw1_ref, pw2_ref, pb1_ref,
             pb2_ref, aw1_ref, aw2_ref, H_ref, A_ref, B_ref, hmin_ref):
    x = x_ref[...]
    t = _dotT(p_ref[...], pw1_ref[...])
    q = _dotT(t, pw2_ref[...])
    u = _dotT(x, wsrc_ref[...]) + q
    H = _dotT(_dotT(u, aw1_ref[...]), aw2_ref[...])
    bp = _dotT(pb1_ref[...], pw2_ref[...]) + pb2_ref[...]
    H_ref[...] = H
    A_ref[...] = _dotT(x, wlin_ref[...]) - q
    B_ref[...] = q + bp
    hmin_ref[0] = jnp.min(H, axis=0, keepdims=True)


def _run_k1(x, pos_pad, W_lin, W_src, pw1_pad, pos_w2, pb1, pb2, attn_w1, attn_w2):
    full = lambda s: pl.BlockSpec(s, lambda i: (0, 0))
    row = pl.BlockSpec((BLK, D), lambda i: (i, 0))
    return pl.pallas_call(
        _k1_body,
        grid=(NB,),
        in_specs=[row, row, full((D, D)), full((D, D)), full((64, D)),
                  full((D, 64)), full((1, 64)), full((1, D)),
                  full((64, D)), full((D, 64))],
        out_specs=[row, row, row, pl.BlockSpec((1, 1, D), lambda i: (i, 0, 0))],
        out_shape=[jax.ShapeDtypeStruct((N, D), jnp.float32),
                   jax.ShapeDtypeStruct((N, D), jnp.float32),
                   jax.ShapeDtypeStruct((N, D), jnp.float32),
                   jax.ShapeDtypeStruct((NB, 1, D), jnp.float32)],
    )(x, pos_pad, W_lin, W_src, pw1_pad, pos_w2, pb1, pb2, attn_w1, attn_w2)


def _k2_body(hpart_ref, H_ref, A_ref, T_ref):
    hmin = jnp.min(hpart_ref[...], axis=0, keepdims=True)
    w = jnp.exp(hmin - H_ref[...])
    T_ref[0] = w
    T_ref[1] = w * A_ref[...]


def _run_k2(hpart, H, A):
    row = pl.BlockSpec((BLK, D), lambda i: (i, 0))
    return pl.pallas_call(
        _k2_body,
        grid=(NB,),
        in_specs=[pl.BlockSpec((NB, D), lambda i: (0, 0)), row, row],
        out_specs=pl.BlockSpec((2, BLK, D), lambda i: (0, i, 0)),
        out_shape=jax.ShapeDtypeStruct((2, N, D), jnp.float32),
    )(hpart, H, A)


CPG = 32


def _make_sc_kernel(n_groups):
    mesh = plsc.VectorSubcoreMesh(core_axis_name="c", subcore_axis_name="s",
                                  num_cores=NC, num_subcores=NS)
    rows_per_tile = ACC_ROWS // NS

    @functools.partial(
        pl.kernel,
        out_type=jax.ShapeDtypeStruct((NC, ACC_ROWS, D), jnp.float32),
        mesh=mesh,
        scratch_types=[
            pltpu.VMEM((CPG, CHUNK), jnp.int32),
            pltpu.VMEM((CPG, CHUNK), jnp.int32),
            pltpu.VMEM((2, CHUNK, D), jnp.float32),
            pltpu.MemorySpace.VMEM_SHARED((ACC_ROWS, D), jnp.float32),
            pltpu.SemaphoreType.DMA,
            pltpu.SemaphoreType.DMA,
        ],
    )
    def sc_scatter(tall_hbm, srcc_hbm, dstr_hbm, acc_hbm,
                   idx_s, idx_d, buf, acc_sh, sem0, sem1):
        c = lax.axis_index("c")
        s = lax.axis_index("s")
        sems = (sem0, sem1)

        def _z(i, carry):
            buf[0, i // 8, pl.ds((i % 8) * 16, 16)] = jnp.zeros((16,), jnp.float32)
            return carry
        lax.fori_loop(0, CHUNK * 8, _z, 0)
        base = s * rows_per_tile
        for k in range(rows_per_tile // CHUNK):
            pltpu.sync_copy(buf.at[0], acc_sh.at[pl.ds(base + k * CHUNK, CHUNK)])
        plsc.subcore_barrier()

        def _group(g, carry):
            pltpu.sync_copy(srcc_hbm.at[c, s, g], idx_s)
            pltpu.sync_copy(dstr_hbm.at[s, g], idx_d)
            pending = pltpu.async_copy(tall_hbm.at[idx_s.at[0]], buf.at[0], sem0)
            for j in range(CPG):
                if j + 1 < CPG:
                    nxt = pltpu.async_copy(tall_hbm.at[idx_s.at[j + 1]],
                                           buf.at[(j + 1) % 2], sems[(j + 1) % 2])
                pending.wait()
                pltpu.sync_copy(buf.at[j % 2], acc_sh.at[idx_d.at[j]], add=True)
                if j + 1 < CPG:
                    pending = nxt
            return carry
        lax.fori_loop(0, n_groups, _group, 0)
        plsc.subcore_barrier()

        pltpu.sync_copy(acc_sh.at[pl.ds(base, rows_per_tile)],
                        acc_hbm.at[c, pl.ds(base, rows_per_tile)])

    return sc_scatter


def _k3_body(a0_ref, a1_ref, t0_ref, t1_ref, b_ref, o_ref, ps_ref, pq_ref):
    denom = a0_ref[...] + t0_ref[...]
    numer = a1_ref[...] + t1_ref[...] + b_ref[...] * denom
    o = numer / (denom + 1e-16)
    o = jnp.where(o > 0, o, jnp.exp(o) - 1.0)
    o_ref[...] = o
    ps_ref[0] = jnp.sum(o, axis=0, keepdims=True)
    pq_ref[0] = jnp.sum(o * o, axis=0, keepdims=True)


def _run_k3(a0, a1, t0, t1, B):
    row = pl.BlockSpec((BLK, D), lambda i: (i, 0))
    return pl.pallas_call(
        _k3_body,
        grid=(NB,),
        in_specs=[row, row, row, row, row],
        out_specs=[row, pl.BlockSpec((1, 1, D), lambda i: (i, 0, 0)),
                   pl.BlockSpec((1, 1, D), lambda i: (i, 0, 0))],
        out_shape=[jax.ShapeDtypeStruct((N, D), jnp.float32),
                   jax.ShapeDtypeStruct((NB, 1, D), jnp.float32),
                   jax.ShapeDtypeStruct((NB, 1, D), jnp.float32)],
    )(a0, a1, t0, t1, B)


def _k4_body(o_ref, ps_ref, pq_ref, g_ref, bb_ref, out_ref):
    inv_n = 1.0 / N
    mean = jnp.sum(ps_ref[...], axis=0, keepdims=True) * inv_n
    var = jnp.sum(pq_ref[...], axis=0, keepdims=True) * inv_n - mean * mean
    scale = lax.rsqrt(var + 1e-5) * g_ref[...]
    out_ref[...] = (o_ref[...] - mean) * scale + bb_ref[...]


def _run_k4(o, ps, pq, gamma, beta):
    row = pl.BlockSpec((BLK, D), lambda i: (i, 0))
    part = pl.BlockSpec((NB, D), lambda i: (0, 0))
    vec = pl.BlockSpec((1, D), lambda i: (0, 0))
    return pl.pallas_call(
        _k4_body,
        grid=(NB,),
        in_specs=[row, part, part, vec, vec],
        out_specs=row,
        out_shape=jax.ShapeDtypeStruct((N, D), jnp.float32),
    )(o, ps, pq, gamma, beta)


def kernel(x, pos, edge_index, W_lin, W_src, W_dst,
           pos_w1, pos_b1, pos_w2, pos_b2,
           attn_w1, attn_b1, attn_w2, attn_b2,
           bn_gamma, bn_beta):
    pos_pad = jnp.pad(pos, ((0, 0), (0, D - pos.shape[1])))
    pw1_pad = jnp.pad(pos_w1, ((0, 0), (0, D - pos_w1.shape[1])))
    pb1 = pos_b1.reshape(1, -1)
    pb2 = pos_b2.reshape(1, -1)

    H, A, B, hpart = _run_k1(x, pos_pad, W_lin, W_src, pw1_pad, pos_w2,
                             pb1, pb2, attn_w1, attn_w2)
    T2 = _run_k2(hpart.reshape(NB, D), H, A)
    t_all = T2.reshape(2 * N, D)

    src, dst = edge_index[0], edge_index[1]
    e_in = src.shape[0]
    n_groups = -(-e_in // (NS * CPG * CHUNK))
    ept = n_groups * CPG * CHUNK
    pad = NS * ept - e_in
    dst_eff = jnp.where(src != dst, dst, GARBAGE)
    src_p = jnp.concatenate([src, jnp.zeros((pad,), jnp.int32)])
    dst_p = jnp.concatenate([dst_eff, jnp.full((pad,), GARBAGE, jnp.int32)])
    src_r = src_p.reshape(NS, n_groups, CPG, CHUNK)
    dst_r = dst_p.reshape(NS, n_groups, CPG, CHUNK)
    src_rc = jnp.stack([src_r, src_r + N])

    acc = _make_sc_kernel(n_groups)(t_all, src_rc, dst_r)

    o, ps, pq = _run_k3(acc[0, :N], acc[1, :N], T2[0], T2[1], B)
    return _run_k4(o, ps.reshape(NB, D), pq.reshape(NB, D),
                   bn_gamma.reshape(1, -1), bn_beta.reshape(1, -1))


# speedup vs baseline: 25.4260x; 1.6631x over previous
import functools

import jax
import jax.numpy as jnp
from jax import lax
from jax.experimental import pallas as pl
from jax.experimental.pallas import tpu as pltpu
from jax.experimental.pallas import tpu_sc as plsc

N = 10000
D = 128
NB = 5
BLK = N // NB
NC = 2
NS = 16
CHUNK = 128
ACC_ROWS = 10072
GARBAGE = N

_HI = lax.Precision.HIGHEST


def _dotT(a, w):
    return lax.dot_general(a, w, (((1,), (1,)), ((), ())),
                           precision=_HI, preferred_element_type=jnp.float32)


def _k1_body(x_ref, p_ref, wlin_ref, wsrc_ref, pw1_ref, pw2_ref, pb1_ref,
             pb2_ref, aw1_ref, aw2_ref, H_ref, A_ref, B_ref, hmin_ref):
    x = x_ref[...]
    t = _dotT(p_ref[...], pw1_ref[...])
    q = _dotT(t, pw2_ref[...])
    u = _dotT(x, wsrc_ref[...]) + q
    H = _dotT(_dotT(u, aw1_ref[...]), aw2_ref[...])
    bp = _dotT(pb1_ref[...], pw2_ref[...]) + pb2_ref[...]
    H_ref[...] = H
    A_ref[...] = _dotT(x, wlin_ref[...]) - q
    B_ref[...] = q + bp
    hmin_ref[0] = jnp.min(H, axis=0, keepdims=True)


def _run_k1(x, pos_pad, W_lin, W_src, pw1_pad, pos_w2, pb1, pb2, attn_w1, attn_w2):
    full = lambda s: pl.BlockSpec(s, lambda i: (0, 0))
    row = pl.BlockSpec((BLK, D), lambda i: (i, 0))
    return pl.pallas_call(
        _k1_body,
        grid=(NB,),
        in_specs=[row, row, full((D, D)), full((D, D)), full((64, D)),
                  full((D, 64)), full((1, 64)), full((1, D)),
                  full((64, D)), full((D, 64))],
        out_specs=[row, row, row, pl.BlockSpec((1, 1, D), lambda i: (i, 0, 0))],
        out_shape=[jax.ShapeDtypeStruct((N, D), jnp.float32),
                   jax.ShapeDtypeStruct((N, D), jnp.float32),
                   jax.ShapeDtypeStruct((N, D), jnp.float32),
                   jax.ShapeDtypeStruct((NB, 1, D), jnp.float32)],
    )(x, pos_pad, W_lin, W_src, pw1_pad, pos_w2, pb1, pb2, attn_w1, attn_w2)


def _k2_body(hpart_ref, H_ref, A_ref, T_ref):
    hmin = jnp.min(hpart_ref[...], axis=0, keepdims=True)
    w = jnp.exp(hmin - H_ref[...])
    T_ref[0] = w
    T_ref[1] = w * A_ref[...]


def _run_k2(hpart, H, A):
    row = pl.BlockSpec((BLK, D), lambda i: (i, 0))
    return pl.pallas_call(
        _k2_body,
        grid=(NB,),
        in_specs=[pl.BlockSpec((NB, D), lambda i: (0, 0)), row, row],
        out_specs=pl.BlockSpec((2, BLK, D), lambda i: (0, i, 0)),
        out_shape=jax.ShapeDtypeStruct((2, N, D), jnp.float32),
    )(hpart, H, A)


def _make_sc_kernel(n_chunks):
    mesh = plsc.VectorSubcoreMesh(core_axis_name="c", subcore_axis_name="s",
                                  num_cores=NC, num_subcores=NS)
    nblk, brem = divmod(ACC_ROWS, CHUNK)
    blk_iters = -(-(nblk + (1 if brem else 0)) // NS)

    @functools.partial(
        pl.kernel,
        out_type=jax.ShapeDtypeStruct((NC, ACC_ROWS, D), jnp.float32),
        mesh=mesh,
        scratch_types=[
            pltpu.VMEM((5, 2, CHUNK), jnp.int32),
            pltpu.VMEM((3, CHUNK, D), jnp.float32),
            pltpu.MemorySpace.VMEM_SHARED((ACC_ROWS, D), jnp.float32),
        ] + [pltpu.SemaphoreType.DMA] * 10,
    )
    def sc_scatter(tall_hbm, idxpk_hbm, acc_hbm, idxr, buf, acc_sh,
                   si0, si1, si2, si3, si4, sg0, sg1, sg2, ss0, ss1):
        c = lax.axis_index("c")
        s = lax.axis_index("s")
        SI = (si0, si1, si2, si3, si4)
        SG = (sg0, sg1, sg2)
        SS = (ss0, ss1)

        def _z(i, carry):
            buf[0, i // 8, pl.ds((i % 8) * 16, 16)] = jnp.zeros((16,), jnp.float32)
            return carry
        lax.fori_loop(0, CHUNK * 8, _z, 0)
        for i in range(blk_iters):
            b = s + NS * i
            @pl.when(b < nblk)
            def _():
                pltpu.sync_copy(buf.at[0], acc_sh.at[pl.ds(b * CHUNK, CHUNK)])
            if brem:
                @pl.when(b == nblk)
                def _():
                    pltpu.sync_copy(buf.at[0, pl.ds(0, brem)],
                                    acc_sh.at[pl.ds(nblk * CHUNK, brem)])
        plsc.subcore_barrier()

        def _lidx(m):
            return pltpu.async_copy(idxpk_hbm.at[c, s, m], idxr.at[m % 5],
                                    SI[m % 5])

        def _gather(m):
            return pltpu.async_copy(tall_hbm.at[idxr.at[m % 5, 0]],
                                    buf.at[m % 3], SG[m % 3])

        idd, gd, sd = {}, {}, {}
        for m in range(min(4, n_chunks)):
            idd[m] = _lidx(m)
        for m in range(min(2, n_chunks)):
            idd[m].wait()
            gd[m] = _gather(m)
        for k in range(n_chunks):
            if k >= 1:
                sd[k - 1].wait()
            if k + 4 < n_chunks:
                idd[k + 4] = _lidx(k + 4)
            if k + 2 < n_chunks:
                idd[k + 2].wait()
                gd[k + 2] = _gather(k + 2)
            gd[k].wait()
            sd[k] = pltpu.async_copy(buf.at[k % 3],
                                     acc_sh.at[idxr.at[k % 5, 1]],
                                     SS[k % 2], add=True)
        sd[n_chunks - 1].wait()
        plsc.subcore_barrier()

        for i in range(blk_iters):
            b = s + NS * i
            @pl.when(b < nblk)
            def _():
                pltpu.sync_copy(acc_sh.at[pl.ds(b * CHUNK, CHUNK)],
                                acc_hbm.at[c, pl.ds(b * CHUNK, CHUNK)])
            if brem:
                @pl.when(b == nblk)
                def _():
                    pltpu.sync_copy(acc_sh.at[pl.ds(nblk * CHUNK, brem)],
                                    acc_hbm.at[c, pl.ds(nblk * CHUNK, brem)])

    return sc_scatter


def _k3_body(a0_ref, a1_ref, t0_ref, t1_ref, b_ref, o_ref, ps_ref, pq_ref):
    denom = a0_ref[...] + t0_ref[...]
    numer = a1_ref[...] + t1_ref[...] + b_ref[...] * denom
    o = numer / (denom + 1e-16)
    o = jnp.where(o > 0, o, jnp.exp(o) - 1.0)
    o_ref[...] = o
    ps_ref[0] = jnp.sum(o, axis=0, keepdims=True)
    pq_ref[0] = jnp.sum(o * o, axis=0, keepdims=True)


def _run_k3(a0, a1, t0, t1, B):
    row = pl.BlockSpec((BLK, D), lambda i: (i, 0))
    return pl.pallas_call(
        _k3_body,
        grid=(NB,),
        in_specs=[row, row, row, row, row],
        out_specs=[row, pl.BlockSpec((1, 1, D), lambda i: (i, 0, 0)),
                   pl.BlockSpec((1, 1, D), lambda i: (i, 0, 0))],
        out_shape=[jax.ShapeDtypeStruct((N, D), jnp.float32),
                   jax.ShapeDtypeStruct((NB, 1, D), jnp.float32),
                   jax.ShapeDtypeStruct((NB, 1, D), jnp.float32)],
    )(a0, a1, t0, t1, B)


def _k4_body(o_ref, ps_ref, pq_ref, g_ref, bb_ref, out_ref):
    inv_n = 1.0 / N
    mean = jnp.sum(ps_ref[...], axis=0, keepdims=True) * inv_n
    var = jnp.sum(pq_ref[...], axis=0, keepdims=True) * inv_n - mean * mean
    scale = lax.rsqrt(var + 1e-5) * g_ref[...]
    out_ref[...] = (o_ref[...] - mean) * scale + bb_ref[...]


def _run_k4(o, ps, pq, gamma, beta):
    row = pl.BlockSpec((BLK, D), lambda i: (i, 0))
    part = pl.BlockSpec((NB, D), lambda i: (0, 0))
    vec = pl.BlockSpec((1, D), lambda i: (0, 0))
    return pl.pallas_call(
        _k4_body,
        grid=(NB,),
        in_specs=[row, part, part, vec, vec],
        out_specs=row,
        out_shape=jax.ShapeDtypeStruct((N, D), jnp.float32),
    )(o, ps, pq, gamma, beta)


def kernel(x, pos, edge_index, W_lin, W_src, W_dst,
           pos_w1, pos_b1, pos_w2, pos_b2,
           attn_w1, attn_b1, attn_w2, attn_b2,
           bn_gamma, bn_beta):
    pos_pad = jnp.pad(pos, ((0, 0), (0, D - pos.shape[1])))
    pw1_pad = jnp.pad(pos_w1, ((0, 0), (0, D - pos_w1.shape[1])))
    pb1 = pos_b1.reshape(1, -1)
    pb2 = pos_b2.reshape(1, -1)

    H, A, B, hpart = _run_k1(x, pos_pad, W_lin, W_src, pw1_pad, pos_w2,
                             pb1, pb2, attn_w1, attn_w2)
    T2 = _run_k2(hpart.reshape(NB, D), H, A)
    t_all = T2.reshape(2 * N, D)

    src, dst = edge_index[0], edge_index[1]
    e_in = src.shape[0]
    n_chunks = -(-e_in // (NS * CHUNK))
    ept = n_chunks * CHUNK
    pad = NS * ept - e_in
    dst_eff = jnp.where(src != dst, dst, GARBAGE)
    src_p = jnp.concatenate([src, jnp.zeros((pad,), jnp.int32)])
    dst_p = jnp.concatenate([dst_eff, jnp.full((pad,), GARBAGE, jnp.int32)])
    src_r = src_p.reshape(NS, n_chunks, 1, CHUNK)
    dst_r = dst_p.reshape(NS, n_chunks, 1, CHUNK)
    idxpk = jnp.stack([
        jnp.concatenate([src_r, dst_r], axis=2),
        jnp.concatenate([src_r + N, dst_r], axis=2),
    ])

    acc = _make_sc_kernel(n_chunks)(t_all, idxpk)

    o, ps, pq = _run_k3(acc[0, :N], acc[1, :N], T2[0], T2[1], B)
    return _run_k4(o, ps.reshape(NB, D), pq.reshape(NB, D),
                   bn_gamma.reshape(1, -1), bn_beta.reshape(1, -1))


# speedup vs baseline: 26.1084x; 1.0268x over previous
import functools

import jax
import jax.numpy as jnp
from jax import lax
from jax.experimental import pallas as pl
from jax.experimental.pallas import tpu as pltpu
from jax.experimental.pallas import tpu_sc as plsc

N = 10000
D = 128
NB = 5
BLK = N // NB
NC = 2
NS = 16
CHUNK = 128
ACC_ROWS = 10072
GARBAGE = N

_HI = lax.Precision.HIGHEST


def _dotT(a, w):
    return lax.dot_general(a, w, (((1,), (1,)), ((), ())),
                           precision=_HI, preferred_element_type=jnp.float32)


def _k1_body(x_ref, p_ref, wlin_ref, wsrc_ref, pw1_ref, pw2_ref, pb1_ref,
             pb2_ref, aw1_ref, aw2_ref, H_ref, A_ref, B_ref, hmin_ref):
    x = x_ref[...]
    t = _dotT(p_ref[...], pw1_ref[...])
    q = _dotT(t, pw2_ref[...])
    u = _dotT(x, wsrc_ref[...]) + q
    H = _dotT(_dotT(u, aw1_ref[...]), aw2_ref[...])
    bp = _dotT(pb1_ref[...], pw2_ref[...]) + pb2_ref[...]
    H_ref[...] = H
    A_ref[...] = _dotT(x, wlin_ref[...]) - q
    B_ref[...] = q + bp
    hmin_ref[0] = jnp.min(H, axis=0, keepdims=True)


def _run_k1(x, pos_pad, W_lin, W_src, pw1_pad, pos_w2, pb1, pb2, attn_w1, attn_w2):
    full = lambda s: pl.BlockSpec(s, lambda i: (0, 0))
    row = pl.BlockSpec((BLK, D), lambda i: (i, 0))
    return pl.pallas_call(
        _k1_body,
        grid=(NB,),
        in_specs=[row, row, full((D, D)), full((D, D)), full((64, D)),
                  full((D, 64)), full((1, 64)), full((1, D)),
                  full((64, D)), full((D, 64))],
        out_specs=[row, row, row, pl.BlockSpec((1, 1, D), lambda i: (i, 0, 0))],
        out_shape=[jax.ShapeDtypeStruct((N, D), jnp.float32),
                   jax.ShapeDtypeStruct((N, D), jnp.float32),
                   jax.ShapeDtypeStruct((N, D), jnp.float32),
                   jax.ShapeDtypeStruct((NB, 1, D), jnp.float32)],
    )(x, pos_pad, W_lin, W_src, pw1_pad, pos_w2, pb1, pb2, attn_w1, attn_w2)


def _k2_body(hpart_ref, H_ref, A_ref, T_ref):
    hmin = jnp.min(hpart_ref[...], axis=0, keepdims=True)
    w = jnp.exp(hmin - H_ref[...])
    T_ref[0] = w
    T_ref[1] = w * A_ref[...]


def _run_k2(hpart, H, A):
    row = pl.BlockSpec((BLK, D), lambda i: (i, 0))
    return pl.pallas_call(
        _k2_body,
        grid=(NB,),
        in_specs=[pl.BlockSpec((NB, D), lambda i: (0, 0)), row, row],
        out_specs=pl.BlockSpec((2, BLK, D), lambda i: (0, i, 0)),
        out_shape=jax.ShapeDtypeStruct((2, N, D), jnp.float32),
    )(hpart, H, A)


def _make_sc_kernel(n_chunks):
    mesh = plsc.VectorSubcoreMesh(core_axis_name="c", subcore_axis_name="s",
                                  num_cores=NC, num_subcores=NS)
    nblk, brem = divmod(ACC_ROWS, CHUNK)
    blk_iters = -(-(nblk + (1 if brem else 0)) // NS)

    @functools.partial(
        pl.kernel,
        out_type=jax.ShapeDtypeStruct((NC, ACC_ROWS, D), jnp.float32),
        mesh=mesh,
        scratch_types=[
            pltpu.VMEM((5, 2, CHUNK), jnp.int32),
            pltpu.VMEM((3, CHUNK, D), jnp.float32),
            pltpu.MemorySpace.VMEM_SHARED((ACC_ROWS, D), jnp.float32),
        ] + [pltpu.SemaphoreType.DMA] * 10,
    )
    def sc_scatter(tall_hbm, idxpk_hbm, acc_hbm, idxr, buf, acc_sh,
                   si0, si1, si2, si3, si4, sg0, sg1, sg2, ss0, ss1):
        c = lax.axis_index("c")
        s = lax.axis_index("s")
        SI = (si0, si1, si2, si3, si4)
        SG = (sg0, sg1, sg2)
        SS = (ss0, ss1)

        def _z(i, carry):
            buf[0, i // 8, pl.ds((i % 8) * 16, 16)] = jnp.zeros((16,), jnp.float32)
            return carry
        lax.fori_loop(0, CHUNK * 8, _z, 0)
        for i in range(blk_iters):
            b = s + NS * i
            @pl.when(b < nblk)
            def _():
                pltpu.sync_copy(buf.at[0], acc_sh.at[pl.ds(b * CHUNK, CHUNK)])
            if brem:
                @pl.when(b == nblk)
                def _():
                    pltpu.sync_copy(buf.at[0, pl.ds(0, brem)],
                                    acc_sh.at[pl.ds(nblk * CHUNK, brem)])
        plsc.subcore_barrier()

        def _lidx(m):
            return pltpu.async_copy(idxpk_hbm.at[c, s, m], idxr.at[m % 5],
                                    SI[m % 5])

        def _gather(m):
            return pltpu.async_copy(tall_hbm.at[idxr.at[m % 5, 0]],
                                    buf.at[m % 3], SG[m % 3])

        idd, gd, sd = {}, {}, {}
        for m in range(min(4, n_chunks)):
            idd[m] = _lidx(m)
        for m in range(min(2, n_chunks)):
            idd[m].wait()
            gd[m] = _gather(m)
        for k in range(n_chunks):
            if k >= 1:
                sd[k - 1].wait()
            if k + 4 < n_chunks:
                idd[k + 4] = _lidx(k + 4)
            if k + 2 < n_chunks:
                idd[k + 2].wait()
                gd[k + 2] = _gather(k + 2)
            gd[k].wait()
            sd[k] = pltpu.async_copy(buf.at[k % 3],
                                     acc_sh.at[idxr.at[k % 5, 1]],
                                     SS[k % 2], add=True)
        sd[n_chunks - 1].wait()
        plsc.subcore_barrier()

        for i in range(blk_iters):
            b = s + NS * i
            @pl.when(b < nblk)
            def _():
                pltpu.sync_copy(acc_sh.at[pl.ds(b * CHUNK, CHUNK)],
                                acc_hbm.at[c, pl.ds(b * CHUNK, CHUNK)])
            if brem:
                @pl.when(b == nblk)
                def _():
                    pltpu.sync_copy(acc_sh.at[pl.ds(nblk * CHUNK, brem)],
                                    acc_hbm.at[c, pl.ds(nblk * CHUNK, brem)])

    return sc_scatter


def _k3_body(a0_ref, a1_ref, t0_ref, t1_ref, b_ref, o_ref, ps_ref, pq_ref):
    denom = a0_ref[0] + t0_ref[0]
    numer = a1_ref[0] + t1_ref[0] + b_ref[...] * denom
    o = numer / (denom + 1e-16)
    o = jnp.where(o > 0, o, jnp.exp(o) - 1.0)
    o_ref[...] = o
    ps_ref[0] = jnp.sum(o, axis=0, keepdims=True)
    pq_ref[0] = jnp.sum(o * o, axis=0, keepdims=True)


def _run_k3(acc, T2, B):
    row = pl.BlockSpec((BLK, D), lambda i: (i, 0))
    acc0 = pl.BlockSpec((1, BLK, D), lambda i: (0, i, 0))
    acc1 = pl.BlockSpec((1, BLK, D), lambda i: (1, i, 0))
    return pl.pallas_call(
        _k3_body,
        grid=(NB,),
        in_specs=[acc0, acc1, acc0, acc1, row],
        out_specs=[row, pl.BlockSpec((1, 1, D), lambda i: (i, 0, 0)),
                   pl.BlockSpec((1, 1, D), lambda i: (i, 0, 0))],
        out_shape=[jax.ShapeDtypeStruct((N, D), jnp.float32),
                   jax.ShapeDtypeStruct((NB, 1, D), jnp.float32),
                   jax.ShapeDtypeStruct((NB, 1, D), jnp.float32)],
    )(acc, acc, T2, T2, B)


def _k4_body(o_ref, ps_ref, pq_ref, g_ref, bb_ref, out_ref):
    inv_n = 1.0 / N
    mean = jnp.sum(ps_ref[...], axis=0, keepdims=True) * inv_n
    var = jnp.sum(pq_ref[...], axis=0, keepdims=True) * inv_n - mean * mean
    scale = lax.rsqrt(var + 1e-5) * g_ref[...]
    out_ref[...] = (o_ref[...] - mean) * scale + bb_ref[...]


def _run_k4(o, ps, pq, gamma, beta):
    row = pl.BlockSpec((BLK, D), lambda i: (i, 0))
    part = pl.BlockSpec((NB, D), lambda i: (0, 0))
    vec = pl.BlockSpec((1, D), lambda i: (0, 0))
    return pl.pallas_call(
        _k4_body,
        grid=(NB,),
        in_specs=[row, part, part, vec, vec],
        out_specs=row,
        out_shape=jax.ShapeDtypeStruct((N, D), jnp.float32),
    )(o, ps, pq, gamma, beta)


def kernel(x, pos, edge_index, W_lin, W_src, W_dst,
           pos_w1, pos_b1, pos_w2, pos_b2,
           attn_w1, attn_b1, attn_w2, attn_b2,
           bn_gamma, bn_beta):
    pos_pad = jnp.pad(pos, ((0, 0), (0, D - pos.shape[1])))
    pw1_pad = jnp.pad(pos_w1, ((0, 0), (0, D - pos_w1.shape[1])))
    pb1 = pos_b1.reshape(1, -1)
    pb2 = pos_b2.reshape(1, -1)

    H, A, B, hpart = _run_k1(x, pos_pad, W_lin, W_src, pw1_pad, pos_w2,
                             pb1, pb2, attn_w1, attn_w2)
    T2 = _run_k2(hpart.reshape(NB, D), H, A)
    t_all = T2.reshape(2 * N, D)

    src, dst = edge_index[0], edge_index[1]
    e_in = src.shape[0]
    n_chunks = -(-e_in // (NS * CHUNK))
    ept = n_chunks * CHUNK
    pad = NS * ept - e_in
    dst_eff = jnp.where(src != dst, dst, GARBAGE)
    src_p = jnp.concatenate([src, jnp.zeros((pad,), jnp.int32)])
    dst_p = jnp.concatenate([dst_eff, jnp.full((pad,), GARBAGE, jnp.int32)])
    src_r = src_p.reshape(NS, n_chunks, 1, CHUNK)
    dst_r = dst_p.reshape(NS, n_chunks, 1, CHUNK)
    idxpk = jnp.stack([
        jnp.concatenate([src_r, dst_r], axis=2),
        jnp.concatenate([src_r + N, dst_r], axis=2),
    ])

    acc = _make_sc_kernel(n_chunks)(t_all, idxpk)

    o, ps, pq = _run_k3(acc, T2, B)
    return _run_k4(o, ps.reshape(NB, D), pq.reshape(NB, D),
                   bn_gamma.reshape(1, -1), bn_beta.reshape(1, -1))


# speedup vs baseline: 27.5087x; 1.0536x over previous
import functools

import jax
import jax.numpy as jnp
from jax import lax
from jax.experimental import pallas as pl
from jax.experimental.pallas import tpu as pltpu
from jax.experimental.pallas import tpu_sc as plsc

N = 10000
D = 128
NB = 5
BLK = N // NB
NC = 2
NS = 16
CHUNK = 128
ACC_ROWS = 10072
GARBAGE = N

_HI = lax.Precision.HIGHEST


def _dotT(a, w):
    return lax.dot_general(a, w, (((1,), (1,)), ((), ())),
                           preferred_element_type=jnp.float32)


def _k1_body(x_ref, p_ref, wlin_ref, wsrc_ref, pw1_ref, pw2_ref, pb1_ref,
             pb2_ref, aw1_ref, aw2_ref, H_ref, A_ref, B_ref, hmin_ref):
    x = x_ref[...]
    t = _dotT(p_ref[...], pw1_ref[...])
    q = _dotT(t, pw2_ref[...])
    u = _dotT(x, wsrc_ref[...]) + q
    H = _dotT(_dotT(u, aw1_ref[...]), aw2_ref[...])
    bp = _dotT(pb1_ref[...], pw2_ref[...]) + pb2_ref[...]
    H_ref[...] = H
    A_ref[...] = _dotT(x, wlin_ref[...]) - q
    B_ref[...] = q + bp
    hmin_ref[0] = jnp.min(H, axis=0, keepdims=True)


def _run_k1(x, pos_pad, W_lin, W_src, pw1_pad, pos_w2, pb1, pb2, attn_w1, attn_w2):
    full = lambda s: pl.BlockSpec(s, lambda i: (0, 0))
    row = pl.BlockSpec((BLK, D), lambda i: (i, 0))
    return pl.pallas_call(
        _k1_body,
        grid=(NB,),
        in_specs=[row, row, full((D, D)), full((D, D)), full((64, D)),
                  full((D, 64)), full((1, 64)), full((1, D)),
                  full((64, D)), full((D, 64))],
        out_specs=[row, row, row, pl.BlockSpec((1, 1, D), lambda i: (i, 0, 0))],
        out_shape=[jax.ShapeDtypeStruct((N, D), jnp.float32),
                   jax.ShapeDtypeStruct((N, D), jnp.float32),
                   jax.ShapeDtypeStruct((N, D), jnp.float32),
                   jax.ShapeDtypeStruct((NB, 1, D), jnp.float32)],
    )(x, pos_pad, W_lin, W_src, pw1_pad, pos_w2, pb1, pb2, attn_w1, attn_w2)


def _k2_body(hpart_ref, H_ref, A_ref, T_ref):
    hmin = jnp.min(hpart_ref[...], axis=0, keepdims=True)
    w = jnp.exp(hmin - H_ref[...])
    T_ref[0] = w
    T_ref[1] = w * A_ref[...]


def _run_k2(hpart, H, A):
    row = pl.BlockSpec((BLK, D), lambda i: (i, 0))
    return pl.pallas_call(
        _k2_body,
        grid=(NB,),
        in_specs=[pl.BlockSpec((NB, D), lambda i: (0, 0)), row, row],
        out_specs=pl.BlockSpec((2, BLK, D), lambda i: (0, i, 0)),
        out_shape=jax.ShapeDtypeStruct((2, N, D), jnp.float32),
    )(hpart, H, A)


def _make_sc_kernel(n_chunks):
    mesh = plsc.VectorSubcoreMesh(core_axis_name="c", subcore_axis_name="s",
                                  num_cores=NC, num_subcores=NS)
    nblk, brem = divmod(ACC_ROWS, CHUNK)
    blk_iters = -(-(nblk + (1 if brem else 0)) // NS)

    @functools.partial(
        pl.kernel,
        out_type=jax.ShapeDtypeStruct((NC, ACC_ROWS, D), jnp.float32),
        mesh=mesh,
        scratch_types=[
            pltpu.VMEM((5, 1, CHUNK), jnp.int32),
            pltpu.VMEM((5, 1, CHUNK), jnp.int32),
            pltpu.VMEM((3, CHUNK, D), jnp.float32),
            pltpu.MemorySpace.VMEM_SHARED((ACC_ROWS, D), jnp.float32),
        ] + [pltpu.SemaphoreType.DMA] * 10,
    )
    def sc_scatter(tall_hbm, srcr_hbm, dstr_hbm, acc_hbm, idxr_s, idxr_d,
                   buf, acc_sh, si0, si1, si2, si3, si4, sg0, sg1, sg2,
                   ss0, ss1):
        c = lax.axis_index("c")
        s = lax.axis_index("s")
        SI = (si0, si1, si2, si3, si4)
        SG = (sg0, sg1, sg2)
        SS = (ss0, ss1)

        def _z(i, carry):
            buf[0, i // 8, pl.ds((i % 8) * 16, 16)] = jnp.zeros((16,), jnp.float32)
            return carry
        lax.fori_loop(0, CHUNK * 8, _z, 0)
        for i in range(blk_iters):
            b = s + NS * i
            @pl.when(b < nblk)
            def _():
                pltpu.sync_copy(buf.at[0], acc_sh.at[pl.ds(b * CHUNK, CHUNK)])
            if brem:
                @pl.when(b == nblk)
                def _():
                    pltpu.sync_copy(buf.at[0, pl.ds(0, brem)],
                                    acc_sh.at[pl.ds(nblk * CHUNK, brem)])
        plsc.subcore_barrier()

        def _lidx(m):
            d1 = pltpu.async_copy(srcr_hbm.at[s, m], idxr_s.at[m % 5],
                                  SI[m % 5])
            d2 = pltpu.async_copy(dstr_hbm.at[s, m], idxr_d.at[m % 5],
                                  SI[m % 5])
            return (d1, d2)

        off = jnp.full((16,), c * N, jnp.int32)

        def _fix(m):
            for l in range(CHUNK // 16):
                sl = pl.ds(l * 16, 16)
                idxr_s[m % 5, 0, sl] = idxr_s[m % 5, 0, sl] + off

        def _gather(m):
            return pltpu.async_copy(tall_hbm.at[idxr_s.at[m % 5, 0]],
                                    buf.at[m % 3], SG[m % 3])

        idd, gd, sd = {}, {}, {}
        for m in range(min(4, n_chunks)):
            idd[m] = _lidx(m)
        for m in range(min(2, n_chunks)):
            for d in idd[m]:
                d.wait()
            _fix(m)
            gd[m] = _gather(m)
        for k in range(n_chunks):
            if k >= 1:
                sd[k - 1].wait()
            if k + 4 < n_chunks:
                idd[k + 4] = _lidx(k + 4)
            if k + 2 < n_chunks:
                for d in idd[k + 2]:
                    d.wait()
                _fix(k + 2)
                gd[k + 2] = _gather(k + 2)
            gd[k].wait()
            sd[k] = pltpu.async_copy(buf.at[k % 3],
                                     acc_sh.at[idxr_d.at[k % 5, 0]],
                                     SS[k % 2], add=True)
        sd[n_chunks - 1].wait()
        plsc.subcore_barrier()

        for i in range(blk_iters):
            b = s + NS * i
            @pl.when(b < nblk)
            def _():
                pltpu.sync_copy(acc_sh.at[pl.ds(b * CHUNK, CHUNK)],
                                acc_hbm.at[c, pl.ds(b * CHUNK, CHUNK)])
            if brem:
                @pl.when(b == nblk)
                def _():
                    pltpu.sync_copy(acc_sh.at[pl.ds(nblk * CHUNK, brem)],
                                    acc_hbm.at[c, pl.ds(nblk * CHUNK, brem)])

    return sc_scatter


def _k3_body(a0_ref, a1_ref, t0_ref, t1_ref, b_ref, o_ref, ps_ref, pq_ref):
    denom = a0_ref[0] + t0_ref[0]
    numer = a1_ref[0] + t1_ref[0] + b_ref[...] * denom
    o = numer / (denom + 1e-16)
    o = jnp.where(o > 0, o, jnp.exp(o) - 1.0)
    o_ref[...] = o
    ps_ref[0] = jnp.sum(o, axis=0, keepdims=True)
    pq_ref[0] = jnp.sum(o * o, axis=0, keepdims=True)


def _run_k3(acc, T2, B):
    row = pl.BlockSpec((BLK, D), lambda i: (i, 0))
    acc0 = pl.BlockSpec((1, BLK, D), lambda i: (0, i, 0))
    acc1 = pl.BlockSpec((1, BLK, D), lambda i: (1, i, 0))
    return pl.pallas_call(
        _k3_body,
        grid=(NB,),
        in_specs=[acc0, acc1, acc0, acc1, row],
        out_specs=[row, pl.BlockSpec((1, 1, D), lambda i: (i, 0, 0)),
                   pl.BlockSpec((1, 1, D), lambda i: (i, 0, 0))],
        out_shape=[jax.ShapeDtypeStruct((N, D), jnp.float32),
                   jax.ShapeDtypeStruct((NB, 1, D), jnp.float32),
                   jax.ShapeDtypeStruct((NB, 1, D), jnp.float32)],
    )(acc, acc, T2, T2, B)


def _k4_body(o_ref, ps_ref, pq_ref, g_ref, bb_ref, out_ref):
    inv_n = 1.0 / N
    mean = jnp.sum(ps_ref[...], axis=0, keepdims=True) * inv_n
    var = jnp.sum(pq_ref[...], axis=0, keepdims=True) * inv_n - mean * mean
    scale = lax.rsqrt(var + 1e-5) * g_ref[...]
    out_ref[...] = (o_ref[...] - mean) * scale + bb_ref[...]


def _run_k4(o, ps, pq, gamma, beta):
    row = pl.BlockSpec((BLK, D), lambda i: (i, 0))
    part = pl.BlockSpec((NB, D), lambda i: (0, 0))
    vec = pl.BlockSpec((1, D), lambda i: (0, 0))
    return pl.pallas_call(
        _k4_body,
        grid=(NB,),
        in_specs=[row, part, part, vec, vec],
        out_specs=row,
        out_shape=jax.ShapeDtypeStruct((N, D), jnp.float32),
    )(o, ps, pq, gamma, beta)


def kernel(x, pos, edge_index, W_lin, W_src, W_dst,
           pos_w1, pos_b1, pos_w2, pos_b2,
           attn_w1, attn_b1, attn_w2, attn_b2,
           bn_gamma, bn_beta):
    pos_pad = jnp.pad(pos, ((0, 0), (0, D - pos.shape[1])))
    pw1_pad = jnp.pad(pos_w1, ((0, 0), (0, D - pos_w1.shape[1])))
    pb1 = pos_b1.reshape(1, -1)
    pb2 = pos_b2.reshape(1, -1)

    H, A, B, hpart = _run_k1(x, pos_pad, W_lin, W_src, pw1_pad, pos_w2,
                             pb1, pb2, attn_w1, attn_w2)
    T2 = _run_k2(hpart.reshape(NB, D), H, A)
    t_all = T2.reshape(2 * N, D)

    src, dst = edge_index[0], edge_index[1]
    e_in = src.shape[0]
    n_chunks = -(-e_in // (NS * CHUNK))
    pad = NS * n_chunks * CHUNK - e_in
    dst_eff = jnp.where(src != dst, dst, GARBAGE)
    src_r = jnp.concatenate([src, jnp.zeros((pad,), jnp.int32)])
    dst_r = jnp.concatenate([dst_eff, jnp.full((pad,), GARBAGE, jnp.int32)])
    src_r = src_r.reshape(NS, n_chunks, 1, CHUNK)
    dst_r = dst_r.reshape(NS, n_chunks, 1, CHUNK)

    acc = _make_sc_kernel(n_chunks)(t_all, src_r, dst_r)

    o, ps, pq = _run_k3(acc, T2, B)
    return _run_k4(o, ps.reshape(NB, D), pq.reshape(NB, D),
                   bn_gamma.reshape(1, -1), bn_beta.reshape(1, -1))


# speedup vs baseline: 30.3697x; 1.1040x over previous
import functools

import jax
import jax.numpy as jnp
from jax import lax
from jax.experimental import pallas as pl
from jax.experimental.pallas import tpu as pltpu
from jax.experimental.pallas import tpu_sc as plsc

N = 10000
D = 128
NB = 5
BLK = N // NB
NC = 2
NS = 16
CHUNK = 128
ACC_ROWS = 10072
GARBAGE = N

_HI = lax.Precision.HIGHEST


def _dotT(a, w):
    return lax.dot_general(a, w, (((1,), (1,)), ((), ())),
                           preferred_element_type=jnp.float32)


def _k1_body(x_ref, p_ref, wlin_ref, wsrc_ref, pw1_ref, pw2_ref, pb1_ref,
             pb2_ref, aw1_ref, aw2_ref, H_ref, A_ref, B_ref, T_ref, hmin_ref):
    i = pl.program_id(0)

    @pl.when(i < NB)
    def _():
        x = x_ref[...]
        t = _dotT(p_ref[...], pw1_ref[...])
        q = _dotT(t, pw2_ref[...])
        u = _dotT(x, wsrc_ref[...]) + q
        H = _dotT(_dotT(u, aw1_ref[...]), aw2_ref[...])
        bp = _dotT(pb1_ref[...], pw2_ref[...]) + pb2_ref[...]
        H_ref[...] = H
        A_ref[...] = _dotT(x, wlin_ref[...]) - q
        B_ref[...] = q + bp
        hmin_ref[pl.ds(i, 1), :] = jnp.min(H, axis=0, keepdims=True)

    @pl.when(i >= NB)
    def _():
        hmin = jnp.min(hmin_ref[pl.ds(0, NB), :], axis=0, keepdims=True)
        w = jnp.exp(hmin - x_ref[...])
        T_ref[0] = w
        T_ref[1] = w * p_ref[...]


def _run_k1(x, pos_pad, W_lin, W_src, pw1_pad, pos_w2, pb1, pb2, attn_w1, attn_w2):
    full = lambda s: pl.BlockSpec(s, lambda i: (0, 0))
    ph = lambda i: jnp.where(i < NB, i, i - NB)
    ph0 = lambda i: jnp.minimum(i, NB - 1)
    row = pl.BlockSpec((BLK, D), lambda i: (ph(i), 0))
    row0 = pl.BlockSpec((BLK, D), lambda i: (ph0(i), 0))
    rowf = pl.BlockSpec((BLK, D), lambda i: (jnp.where(i < NB, i, 0), 0))
    return pl.pallas_call(
        _k1_body,
        grid=(2 * NB,),
        in_specs=[row, row, full((D, D)), full((D, D)), full((64, D)),
                  full((D, 64)), full((1, 64)), full((1, D)),
                  full((64, D)), full((D, 64))],
        out_specs=[rowf, rowf, row0,
                   pl.BlockSpec((2, BLK, D), lambda i: (0, jnp.maximum(i - NB, 0), 0))],
        out_shape=[jax.ShapeDtypeStruct((N, D), jnp.float32),
                   jax.ShapeDtypeStruct((N, D), jnp.float32),
                   jax.ShapeDtypeStruct((N, D), jnp.float32),
                   jax.ShapeDtypeStruct((2, N, D), jnp.float32)],
        scratch_shapes=[pltpu.VMEM((NB, D), jnp.float32)],
        input_output_aliases={0: 0, 1: 1},
    )(x, pos_pad, W_lin, W_src, pw1_pad, pos_w2, pb1, pb2, attn_w1, attn_w2)


def _make_sc_kernel(n_chunks):
    mesh = plsc.VectorSubcoreMesh(core_axis_name="c", subcore_axis_name="s",
                                  num_cores=NC, num_subcores=NS)
    nblk, brem = divmod(ACC_ROWS, CHUNK)
    blk_iters = -(-(nblk + (1 if brem else 0)) // NS)

    @functools.partial(
        pl.kernel,
        out_type=jax.ShapeDtypeStruct((NC, ACC_ROWS, D), jnp.float32),
        mesh=mesh,
        scratch_types=[
            pltpu.VMEM((5, 1, CHUNK), jnp.int32),
            pltpu.VMEM((5, 1, CHUNK), jnp.int32),
            pltpu.VMEM((3, CHUNK, D), jnp.float32),
            pltpu.MemorySpace.VMEM_SHARED((ACC_ROWS, D), jnp.float32),
        ] + [pltpu.SemaphoreType.DMA] * 10,
    )
    def sc_scatter(tall_hbm, srcr_hbm, dstr_hbm, acc_hbm, idxr_s, idxr_d,
                   buf, acc_sh, si0, si1, si2, si3, si4, sg0, sg1, sg2,
                   ss0, ss1):
        c = lax.axis_index("c")
        s = lax.axis_index("s")
        SI = (si0, si1, si2, si3, si4)
        SG = (sg0, sg1, sg2)
        SS = (ss0, ss1)

        def _z(i, carry):
            buf[0, i // 8, pl.ds((i % 8) * 16, 16)] = jnp.zeros((16,), jnp.float32)
            return carry
        lax.fori_loop(0, CHUNK * 8, _z, 0)
        for i in range(blk_iters):
            b = s + NS * i
            @pl.when(b < nblk)
            def _():
                pltpu.sync_copy(buf.at[0], acc_sh.at[pl.ds(b * CHUNK, CHUNK)])
            if brem:
                @pl.when(b == nblk)
                def _():
                    pltpu.sync_copy(buf.at[0, pl.ds(0, brem)],
                                    acc_sh.at[pl.ds(nblk * CHUNK, brem)])
        plsc.subcore_barrier()

        def _lidx(m):
            d1 = pltpu.async_copy(srcr_hbm.at[s, m], idxr_s.at[m % 5],
                                  SI[m % 5])
            d2 = pltpu.async_copy(dstr_hbm.at[s, m], idxr_d.at[m % 5],
                                  SI[m % 5])
            return (d1, d2)

        off = jnp.full((16,), c * N, jnp.int32)

        def _fix(m):
            for l in range(CHUNK // 16):
                sl = pl.ds(l * 16, 16)
                idxr_s[m % 5, 0, sl] = idxr_s[m % 5, 0, sl] + off

        def _gather(m):
            return pltpu.async_copy(tall_hbm.at[idxr_s.at[m % 5, 0]],
                                    buf.at[m % 3], SG[m % 3])

        idd, gd, sd = {}, {}, {}
        for m in range(min(4, n_chunks)):
            idd[m] = _lidx(m)
        for m in range(min(2, n_chunks)):
            for d in idd[m]:
                d.wait()
            _fix(m)
            gd[m] = _gather(m)
        for k in range(n_chunks):
            if k >= 1:
                sd[k - 1].wait()
            if k + 4 < n_chunks:
                idd[k + 4] = _lidx(k + 4)
            if k + 2 < n_chunks:
                for d in idd[k + 2]:
                    d.wait()
                _fix(k + 2)
                gd[k + 2] = _gather(k + 2)
            gd[k].wait()
            sd[k] = pltpu.async_copy(buf.at[k % 3],
                                     acc_sh.at[idxr_d.at[k % 5, 0]],
                                     SS[k % 2], add=True)
        sd[n_chunks - 1].wait()
        plsc.subcore_barrier()

        for i in range(blk_iters):
            b = s + NS * i
            @pl.when(b < nblk)
            def _():
                pltpu.sync_copy(acc_sh.at[pl.ds(b * CHUNK, CHUNK)],
                                acc_hbm.at[c, pl.ds(b * CHUNK, CHUNK)])
            if brem:
                @pl.when(b == nblk)
                def _():
                    pltpu.sync_copy(acc_sh.at[pl.ds(nblk * CHUNK, brem)],
                                    acc_hbm.at[c, pl.ds(nblk * CHUNK, brem)])

    return sc_scatter


def _k3_body(a0_ref, a1_ref, t0_ref, t1_ref, b_ref, oin_ref, g_ref, bb_ref,
             o_ref, psum_ref):
    i = pl.program_id(0)

    @pl.when(i < NB)
    def _():
        denom = a0_ref[0] + t0_ref[0]
        numer = a1_ref[0] + t1_ref[0] + b_ref[...] * denom
        o = numer / (denom + 1e-16)
        o = jnp.where(o > 0, o, jnp.exp(o) - 1.0)
        o_ref[...] = o
        psum_ref[pl.ds(i, 1), :] = jnp.sum(o, axis=0, keepdims=True)
        psum_ref[pl.ds(NB + i, 1), :] = jnp.sum(o * o, axis=0, keepdims=True)

    @pl.when(i >= NB)
    def _():
        inv_n = 1.0 / N
        mean = jnp.sum(psum_ref[pl.ds(0, NB), :], axis=0, keepdims=True) * inv_n
        sq = jnp.sum(psum_ref[pl.ds(NB, NB), :], axis=0, keepdims=True) * inv_n
        var = sq - mean * mean
        scale = lax.rsqrt(var + 1e-5) * g_ref[...]
        o_ref[...] = (oin_ref[...] - mean) * scale + bb_ref[...]


def _run_k3(acc, T2, B, donor, gamma, beta):
    blk = lambda i: jnp.minimum(i, NB - 1)
    row = pl.BlockSpec((BLK, D), lambda i: (blk(i), 0))
    acc0 = pl.BlockSpec((1, BLK, D), lambda i: (0, blk(i), 0))
    acc1 = pl.BlockSpec((1, BLK, D), lambda i: (1, blk(i), 0))
    ph = lambda i: jnp.where(i < NB, i, i - NB)
    orow = pl.BlockSpec((BLK, D), lambda i: (ph(i), 0))
    oin = pl.BlockSpec((BLK, D), lambda i: (jnp.where(i < NB, NB - 1, i - NB), 0))
    vec = pl.BlockSpec((1, D), lambda i: (0, 0))
    return pl.pallas_call(
        _k3_body,
        grid=(2 * NB,),
        in_specs=[acc0, acc1, acc0, acc1, row, oin, vec, vec],
        out_specs=orow,
        out_shape=jax.ShapeDtypeStruct((N, D), jnp.float32),
        scratch_shapes=[pltpu.VMEM((2 * NB, D), jnp.float32)],
        input_output_aliases={5: 0},
    )(acc, acc, T2, T2, B, donor, gamma, beta)


def kernel(x, pos, edge_index, W_lin, W_src, W_dst,
           pos_w1, pos_b1, pos_w2, pos_b2,
           attn_w1, attn_b1, attn_w2, attn_b2,
           bn_gamma, bn_beta):
    pos_pad = jnp.pad(pos, ((0, 0), (0, D - pos.shape[1])))
    pw1_pad = jnp.pad(pos_w1, ((0, 0), (0, D - pos_w1.shape[1])))
    pb1 = pos_b1.reshape(1, -1)
    pb2 = pos_b2.reshape(1, -1)

    H, A, B, T2 = _run_k1(x, pos_pad, W_lin, W_src, pw1_pad, pos_w2,
                          pb1, pb2, attn_w1, attn_w2)
    t_all = T2.reshape(2 * N, D)

    src, dst = edge_index[0], edge_index[1]
    e_in = src.shape[0]
    n_chunks = -(-e_in // (NS * CHUNK))
    pad = NS * n_chunks * CHUNK - e_in
    dst_eff = jnp.where(src != dst, dst, GARBAGE)
    src_r = jnp.concatenate([src, jnp.zeros((pad,), jnp.int32)])
    dst_r = jnp.concatenate([dst_eff, jnp.full((pad,), GARBAGE, jnp.int32)])
    src_r = src_r.reshape(NS, n_chunks, 1, CHUNK)
    dst_r = dst_r.reshape(NS, n_chunks, 1, CHUNK)

    acc = _make_sc_kernel(n_chunks)(t_all, src_r, dst_r)

    return _run_k3(acc, T2, B, H,
                   bn_gamma.reshape(1, -1), bn_beta.reshape(1, -1))
